# baseline (speedup 1.0000x reference)
import jax
import jax.numpy as jnp
from jax.experimental import pallas as pl
from jax.experimental.pallas import tpu as pltpu

K = 20
EPS = 1e-5
B = 8
N = 1024
NEG = -3.0e38


def _lrelu(v):
    return jnp.where(v >= 0, v, 0.2 * v)


def _gcn_body(x_ref, w1, w2, w3, w4, w5,
              s1, t1, s2, t2, s3, t3, s4, t4, s5, t5, out_ref):
    xb = x_ref[0]
    g = jax.lax.dot_general(xb, xb, (((0,), (0,)), ((), ())),
                            preferred_element_type=jnp.float32)
    xx = jnp.sum(xb * xb, axis=0)
    pd = 2.0 * g - xx[:, None] - xx[None, :]

    lane = jax.lax.broadcasted_iota(jnp.int32, (N, N), 1)

    def step(_, carry):
        pd_c, a_c = carry
        m = jnp.max(pd_c, axis=1, keepdims=True)
        cand = pd_c == m
        fi = jnp.min(jnp.where(cand, lane, N), axis=1, keepdims=True)
        first = lane == fi
        return jnp.where(first, NEG, pd_c), jnp.where(first, 1.0, a_c)

    _, a = jax.lax.fori_loop(0, K, step, (pd, jnp.zeros((N, N), jnp.float32)))

    sub = jax.lax.broadcasted_iota(jnp.int32, (N, N), 0)
    a = jnp.where(lane == sub, 1.0, a)
    dinv = jax.lax.rsqrt(jnp.sum(a, axis=0))
    ab = a.astype(jnp.bfloat16)
    dcol = dinv[:, None]

    def layer(hw, s, t):
        hs = hw * dcol
        hi = hs.astype(jnp.bfloat16)
        lo = (hs - hi.astype(jnp.float32)).astype(jnp.bfloat16)
        agg = jax.lax.dot_general(ab, hi, (((0,), (0,)), ((), ())),
                                  preferred_element_type=jnp.float32)
        agg += jax.lax.dot_general(ab, lo, (((0,), (0,)), ((), ())),
                                   preferred_element_type=jnp.float32)
        return _lrelu(agg * (dcol * s[...]) + t[...])

    hw = jax.lax.dot_general(xb, w1[...], (((0,), (0,)), ((), ())),
                             preferred_element_type=jnp.float32)
    h = layer(hw, s1, t1)
    h = layer(jnp.dot(h, w2[...], preferred_element_type=jnp.float32), s2, t2)
    h = layer(jnp.dot(h, w3[...], preferred_element_type=jnp.float32), s3, t3)
    h = layer(jnp.dot(h, w4[...], preferred_element_type=jnp.float32), s4, t4)
    h = layer(jnp.dot(h, w5[...], preferred_element_type=jnp.float32), s5, t5)
    out_ref[0, 0] = jnp.sum(h, axis=0)


def _head_body(s_ref, l1, s6, t6, l2, s7, t7, l3, t8, out_ref):
    s = s_ref[...]
    y = (jnp.dot(s * (1.0 / N), l1[:N, :],
                 preferred_element_type=jnp.float32)
         + jnp.dot(s, l1[N:, :], preferred_element_type=jnp.float32))
    y = _lrelu(y * s6[...] + t6[...])
    y = _lrelu(jnp.dot(y, l2[...], preferred_element_type=jnp.float32)
               * s7[...] + t7[...])
    out_ref[...] = jnp.dot(y, l3[...], preferred_element_type=jnp.float32) + t8[...]


def kernel(x, W1, b1, W2, b2, W3, b3, W4, b4, W5, b5,
           g1, be1, g2, be2, g3, be3, g4, be4, g5, be5, g6, be6, g7, be7,
           L1W, L2W, L2b, L3W, L3b):
    inv = jnp.float32(1.0 / jnp.sqrt(1.0 + EPS))

    def fuse(gv, bev, bv=None):
        s = (gv * inv).reshape(1, -1)
        t = (bev if bv is None else bv * gv * inv + bev).reshape(1, -1)
        return s, t

    s1, t1 = fuse(g1, be1, b1)
    s2, t2 = fuse(g2, be2, b2)
    s3, t3 = fuse(g3, be3, b3)
    s4, t4 = fuse(g4, be4, b4)
    s5, t5 = fuse(g5, be5, b5)
    s6, t6 = fuse(g6, be6)
    s7, t7 = fuse(g7, be7, L2b)
    t8 = L3b.reshape(1, -1)

    dims = [64, 128, 256, 512, 1024]
    full = lambda a: pl.BlockSpec(a.shape, lambda b: (0,) * a.ndim)
    vec_specs = []
    for d in dims:
        vec_specs += [pl.BlockSpec((1, d), lambda b: (0, 0))] * 2

    pooled = pl.pallas_call(
        _gcn_body,
        grid=(B,),
        in_specs=[pl.BlockSpec((1, 3, N), lambda b: (b, 0, 0)),
                  full(W1), full(W2), full(W3), full(W4), full(W5)] + vec_specs,
        out_specs=pl.BlockSpec((1, 1, N), lambda b: (b, 0, 0)),
        out_shape=jax.ShapeDtypeStruct((B, 1, N), jnp.float32),
        compiler_params=pltpu.CompilerParams(
            dimension_semantics=("parallel",)),
    )(x, W1, W2, W3, W4, W5, s1, t1, s2, t2, s3, t3, s4, t4, s5, t5)
    pooled = pooled.reshape(B, N)

    out = pl.pallas_call(
        _head_body,
        out_shape=jax.ShapeDtypeStruct((B, 40), jnp.float32),
    )(pooled, L1W, s6, t6, L2W, s7, t7, L3W, t8)
    return out


# speedup vs baseline: 1.1381x; 1.1381x over previous
import jax
import jax.numpy as jnp
from jax.experimental import pallas as pl
from jax.experimental.pallas import tpu as pltpu

K = 20
EPS = 1e-5
B = 8
N = 1024
NEG = -3.0e38


def _lrelu(v):
    return jnp.where(v >= 0, v, 0.2 * v)


def _gcn_body(x_ref, w1, w2, w3, w4, w5,
              s1, t1, s2, t2, s3, t3, s4, t4, s5, t5, out_ref):
    xb = x_ref[0]
    g = jax.lax.dot_general(xb, xb, (((0,), (0,)), ((), ())),
                            preferred_element_type=jnp.float32)
    xx = jnp.sum(xb * xb, axis=0)
    pd = 2.0 * g - xx[:, None] - xx[None, :]

    lane = jax.lax.broadcasted_iota(jnp.int32, (N, N), 1)

    def step(_, carry):
        pd_c, a_c = carry
        m = jnp.max(pd_c, axis=1, keepdims=True)
        cand = pd_c == m
        fi = jnp.min(jnp.where(cand, lane, N), axis=1, keepdims=True)
        first = lane == fi
        return jnp.where(first, NEG, pd_c), jnp.where(first, 1.0, a_c)

    _, a = jax.lax.fori_loop(0, K, step, (pd, jnp.zeros((N, N), jnp.float32)))

    sub = jax.lax.broadcasted_iota(jnp.int32, (N, N), 0)
    a = jnp.where(lane == sub, 1.0, a)
    dinv = jax.lax.rsqrt(jnp.sum(a, axis=0))
    ab = a.astype(jnp.bfloat16)
    dcol = dinv[:, None]

    def layer(hw, s, t):
        hs = hw * dcol
        hi = hs.astype(jnp.bfloat16)
        agg = jax.lax.dot_general(ab, hi, (((0,), (0,)), ((), ())),
                                  preferred_element_type=jnp.float32)
        return _lrelu(agg * (dcol * s[...]) + t[...])

    hw = jax.lax.dot_general(xb, w1[...], (((0,), (0,)), ((), ())),
                             preferred_element_type=jnp.float32)
    h = layer(hw, s1, t1)
    h = layer(jnp.dot(h, w2[...], preferred_element_type=jnp.float32), s2, t2)
    h = layer(jnp.dot(h, w3[...], preferred_element_type=jnp.float32), s3, t3)
    h = layer(jnp.dot(h, w4[...], preferred_element_type=jnp.float32), s4, t4)
    h = layer(jnp.dot(h, w5[...], preferred_element_type=jnp.float32), s5, t5)
    out_ref[0, 0] = jnp.sum(h, axis=0)


def _head_body(s_ref, l1, s6, t6, l2, s7, t7, l3, t8, out_ref):
    s = s_ref[...]
    y = (jnp.dot(s * (1.0 / N), l1[:N, :],
                 preferred_element_type=jnp.float32)
         + jnp.dot(s, l1[N:, :], preferred_element_type=jnp.float32))
    y = _lrelu(y * s6[...] + t6[...])
    y = _lrelu(jnp.dot(y, l2[...], preferred_element_type=jnp.float32)
               * s7[...] + t7[...])
    out_ref[...] = jnp.dot(y, l3[...], preferred_element_type=jnp.float32) + t8[...]


def kernel(x, W1, b1, W2, b2, W3, b3, W4, b4, W5, b5,
           g1, be1, g2, be2, g3, be3, g4, be4, g5, be5, g6, be6, g7, be7,
           L1W, L2W, L2b, L3W, L3b):
    inv = jnp.float32(1.0 / jnp.sqrt(1.0 + EPS))

    def fuse(gv, bev, bv=None):
        s = (gv * inv).reshape(1, -1)
        t = (bev if bv is None else bv * gv * inv + bev).reshape(1, -1)
        return s, t

    s1, t1 = fuse(g1, be1, b1)
    s2, t2 = fuse(g2, be2, b2)
    s3, t3 = fuse(g3, be3, b3)
    s4, t4 = fuse(g4, be4, b4)
    s5, t5 = fuse(g5, be5, b5)
    s6, t6 = fuse(g6, be6)
    s7, t7 = fuse(g7, be7, L2b)
    t8 = L3b.reshape(1, -1)

    dims = [64, 128, 256, 512, 1024]
    full = lambda a: pl.BlockSpec(a.shape, lambda b: (0,) * a.ndim)
    vec_specs = []
    for d in dims:
        vec_specs += [pl.BlockSpec((1, d), lambda b: (0, 0))] * 2

    pooled = pl.pallas_call(
        _gcn_body,
        grid=(B,),
        in_specs=[pl.BlockSpec((1, 3, N), lambda b: (b, 0, 0)),
                  full(W1), full(W2), full(W3), full(W4), full(W5)] + vec_specs,
        out_specs=pl.BlockSpec((1, 1, N), lambda b: (b, 0, 0)),
        out_shape=jax.ShapeDtypeStruct((B, 1, N), jnp.float32),
        compiler_params=pltpu.CompilerParams(
            dimension_semantics=("parallel",)),
    )(x, W1, W2, W3, W4, W5, s1, t1, s2, t2, s3, t3, s4, t4, s5, t5)
    pooled = pooled.reshape(B, N)

    out = pl.pallas_call(
        _head_body,
        out_shape=jax.ShapeDtypeStruct((B, 40), jnp.float32),
    )(pooled, L1W, s6, t6, L2W, s7, t7, L3W, t8)
    return out


# speedup vs baseline: 1.5533x; 1.3647x over previous
import jax
import jax.numpy as jnp
from jax.experimental import pallas as pl
from jax.experimental.pallas import tpu as pltpu

K = 20
EPS = 1e-5
B = 8
N = 1024
NEG = -3.0e38


def _lrelu(v):
    return jnp.where(v >= 0, v, 0.2 * v)


def _gcn_body(x_ref, w1, w2, w3, w4, w5,
              s1, t1, s2, t2, s3, t3, s4, t4, s5, t5, out_ref):
    xb = x_ref[0]
    g = jax.lax.dot_general(xb, xb, (((0,), (0,)), ((), ())),
                            preferred_element_type=jnp.float32)
    xx = jnp.sum(xb * xb, axis=0)
    pd = 2.0 * g - xx[:, None] - xx[None, :]

    lane = jax.lax.broadcasted_iota(jnp.int32, (N, N), 1)

    def step(_, pd_c):
        m = jnp.max(pd_c, axis=1, keepdims=True)
        fi = jnp.min(jnp.where(pd_c == m, lane, N), axis=1, keepdims=True)
        return jnp.where(lane == fi, NEG, pd_c)

    pdf = jax.lax.fori_loop(0, K, step, pd)
    a = (pdf == NEG).astype(jnp.float32)

    sub = jax.lax.broadcasted_iota(jnp.int32, (N, N), 0)
    a = jnp.where(lane == sub, 1.0, a)
    dinv = jax.lax.rsqrt(jnp.sum(a, axis=0))
    a_hat = a * dinv[:, None] * dinv[None, :]

    def layer(hw, s, t):
        agg = jax.lax.dot_general(a_hat, hw, (((0,), (0,)), ((), ())),
                                  preferred_element_type=jnp.float32)
        return _lrelu(agg * s[...] + t[...])

    hw = jax.lax.dot_general(xb, w1[...], (((0,), (0,)), ((), ())),
                             preferred_element_type=jnp.float32)
    h = layer(hw, s1, t1)
    h = layer(jnp.dot(h, w2[...], preferred_element_type=jnp.float32), s2, t2)
    h = layer(jnp.dot(h, w3[...], preferred_element_type=jnp.float32), s3, t3)
    h = layer(jnp.dot(h, w4[...], preferred_element_type=jnp.float32), s4, t4)
    h = layer(jnp.dot(h, w5[...], preferred_element_type=jnp.float32), s5, t5)
    out_ref[0, 0] = jnp.sum(h, axis=0)


def _head_body(s_ref, l1, s6, t6, l2, s7, t7, l3, t8, out_ref):
    s = s_ref[...]
    y = (jnp.dot(s * (1.0 / N), l1[:N, :],
                 preferred_element_type=jnp.float32)
         + jnp.dot(s, l1[N:, :], preferred_element_type=jnp.float32))
    y = _lrelu(y * s6[...] + t6[...])
    y = _lrelu(jnp.dot(y, l2[...], preferred_element_type=jnp.float32)
               * s7[...] + t7[...])
    out_ref[...] = jnp.dot(y, l3[...], preferred_element_type=jnp.float32) + t8[...]


def kernel(x, W1, b1, W2, b2, W3, b3, W4, b4, W5, b5,
           g1, be1, g2, be2, g3, be3, g4, be4, g5, be5, g6, be6, g7, be7,
           L1W, L2W, L2b, L3W, L3b):
    inv = jnp.float32(1.0 / jnp.sqrt(1.0 + EPS))

    def fuse(gv, bev, bv=None):
        s = (gv * inv).reshape(1, -1)
        t = (bev if bv is None else bv * gv * inv + bev).reshape(1, -1)
        return s, t

    s1, t1 = fuse(g1, be1, b1)
    s2, t2 = fuse(g2, be2, b2)
    s3, t3 = fuse(g3, be3, b3)
    s4, t4 = fuse(g4, be4, b4)
    s5, t5 = fuse(g5, be5, b5)
    s6, t6 = fuse(g6, be6)
    s7, t7 = fuse(g7, be7, L2b)
    t8 = L3b.reshape(1, -1)

    dims = [64, 128, 256, 512, 1024]
    full = lambda a: pl.BlockSpec(a.shape, lambda b: (0,) * a.ndim)
    vec_specs = []
    for d in dims:
        vec_specs += [pl.BlockSpec((1, d), lambda b: (0, 0))] * 2

    pooled = pl.pallas_call(
        _gcn_body,
        grid=(B,),
        in_specs=[pl.BlockSpec((1, 3, N), lambda b: (b, 0, 0)),
                  full(W1), full(W2), full(W3), full(W4), full(W5)] + vec_specs,
        out_specs=pl.BlockSpec((1, 1, N), lambda b: (b, 0, 0)),
        out_shape=jax.ShapeDtypeStruct((B, 1, N), jnp.float32),
        compiler_params=pltpu.CompilerParams(
            dimension_semantics=("parallel",)),
    )(x, W1, W2, W3, W4, W5, s1, t1, s2, t2, s3, t3, s4, t4, s5, t5)
    pooled = pooled.reshape(B, N)

    out = pl.pallas_call(
        _head_body,
        out_shape=jax.ShapeDtypeStruct((B, 40), jnp.float32),
    )(pooled, L1W, s6, t6, L2W, s7, t7, L3W, t8)
    return out


# speedup vs baseline: 2.0226x; 1.3021x over previous
import jax
import jax.numpy as jnp
from jax.experimental import pallas as pl
from jax.experimental.pallas import tpu as pltpu

K = 20
EPS = 1e-5
B = 8
N = 1024
NEG = -3.0e38


def _lrelu(v):
    return jnp.where(v >= 0, v, 0.2 * v)


def _gcn_body(x_ref, w1, w2, w3, w4, w5,
              s1, t1, s2, t2, s3, t3, s4, t4, s5, t5, out_ref):
    xb = x_ref[0]
    g = jax.lax.dot_general(xb, xb, (((0,), (0,)), ((), ())),
                            preferred_element_type=jnp.float32)
    xx = jnp.sum(xb * xb, axis=0)
    pd = 2.0 * g - xx[:, None] - xx[None, :]

    lane = jax.lax.broadcasted_iota(jnp.int32, (N, N), 1)

    bits = jax.lax.bitcast_convert_type(pd + 0.0, jnp.int32)
    key = bits ^ (jax.lax.shift_right_arithmetic(bits, 31)
                  & jnp.int32(0x7FFFFFFF))

    cnt0 = jnp.sum((key >= 0).astype(jnp.int32), axis=1, keepdims=True)
    t0 = jnp.where(cnt0 >= K, 0, jnp.int32(-2147483648))

    def bstep(i, t_c):
        cand = t_c | jnp.left_shift(jnp.int32(1), 30 - i)
        cnt = jnp.sum((key >= cand).astype(jnp.int32), axis=1, keepdims=True)
        return jnp.where(cnt >= K, cand, t_c)

    t = jax.lax.fori_loop(0, 31, bstep, t0)

    gt = key > t
    eqm = key == t
    need = K - jnp.sum(gt.astype(jnp.int32), axis=1, keepdims=True)
    sub0 = jax.lax.broadcasted_iota(jnp.int32, (N, N), 0)
    ltri = (sub0 < lane).astype(jnp.bfloat16)
    ranks = jax.lax.dot_general(eqm.astype(jnp.bfloat16), ltri,
                                (((1,), (0,)), ((), ())),
                                preferred_element_type=jnp.float32)
    a = (gt | (eqm & (ranks < need.astype(jnp.float32)))).astype(jnp.float32)

    sub = jax.lax.broadcasted_iota(jnp.int32, (N, N), 0)
    a = jnp.where(lane == sub, 1.0, a)
    dinv = jax.lax.rsqrt(jnp.sum(a, axis=0))
    a_hat = a * dinv[:, None] * dinv[None, :]

    def layer(hw, s, t):
        agg = jax.lax.dot_general(a_hat, hw, (((0,), (0,)), ((), ())),
                                  preferred_element_type=jnp.float32)
        return _lrelu(agg * s[...] + t[...])

    hw = jax.lax.dot_general(xb, w1[...], (((0,), (0,)), ((), ())),
                             preferred_element_type=jnp.float32)
    h = layer(hw, s1, t1)
    h = layer(jnp.dot(h, w2[...], preferred_element_type=jnp.float32), s2, t2)
    h = layer(jnp.dot(h, w3[...], preferred_element_type=jnp.float32), s3, t3)
    h = layer(jnp.dot(h, w4[...], preferred_element_type=jnp.float32), s4, t4)
    h = layer(jnp.dot(h, w5[...], preferred_element_type=jnp.float32), s5, t5)
    out_ref[0, 0] = jnp.sum(h, axis=0)


def _head_body(s_ref, l1, s6, t6, l2, s7, t7, l3, t8, out_ref):
    s = s_ref[...]
    y = (jnp.dot(s * (1.0 / N), l1[:N, :],
                 preferred_element_type=jnp.float32)
         + jnp.dot(s, l1[N:, :], preferred_element_type=jnp.float32))
    y = _lrelu(y * s6[...] + t6[...])
    y = _lrelu(jnp.dot(y, l2[...], preferred_element_type=jnp.float32)
               * s7[...] + t7[...])
    out_ref[...] = jnp.dot(y, l3[...], preferred_element_type=jnp.float32) + t8[...]


def kernel(x, W1, b1, W2, b2, W3, b3, W4, b4, W5, b5,
           g1, be1, g2, be2, g3, be3, g4, be4, g5, be5, g6, be6, g7, be7,
           L1W, L2W, L2b, L3W, L3b):
    inv = jnp.float32(1.0 / jnp.sqrt(1.0 + EPS))

    def fuse(gv, bev, bv=None):
        s = (gv * inv).reshape(1, -1)
        t = (bev if bv is None else bv * gv * inv + bev).reshape(1, -1)
        return s, t

    s1, t1 = fuse(g1, be1, b1)
    s2, t2 = fuse(g2, be2, b2)
    s3, t3 = fuse(g3, be3, b3)
    s4, t4 = fuse(g4, be4, b4)
    s5, t5 = fuse(g5, be5, b5)
    s6, t6 = fuse(g6, be6)
    s7, t7 = fuse(g7, be7, L2b)
    t8 = L3b.reshape(1, -1)

    dims = [64, 128, 256, 512, 1024]
    full = lambda a: pl.BlockSpec(a.shape, lambda b: (0,) * a.ndim)
    vec_specs = []
    for d in dims:
        vec_specs += [pl.BlockSpec((1, d), lambda b: (0, 0))] * 2

    pooled = pl.pallas_call(
        _gcn_body,
        grid=(B,),
        in_specs=[pl.BlockSpec((1, 3, N), lambda b: (b, 0, 0)),
                  full(W1), full(W2), full(W3), full(W4), full(W5)] + vec_specs,
        out_specs=pl.BlockSpec((1, 1, N), lambda b: (b, 0, 0)),
        out_shape=jax.ShapeDtypeStruct((B, 1, N), jnp.float32),
        compiler_params=pltpu.CompilerParams(
            dimension_semantics=("parallel",)),
    )(x, W1, W2, W3, W4, W5, s1, t1, s2, t2, s3, t3, s4, t4, s5, t5)
    pooled = pooled.reshape(B, N)

    out = pl.pallas_call(
        _head_body,
        out_shape=jax.ShapeDtypeStruct((B, 40), jnp.float32),
    )(pooled, L1W, s6, t6, L2W, s7, t7, L3W, t8)
    return out
